# baseline (speedup 1.0000x reference)
import functools

import jax
import jax.numpy as jnp
from jax import lax
from jax.experimental import pallas as pl
from jax.experimental.pallas import tpu as pltpu
from jax.experimental.pallas import tpu_sc as plsc

_V = 21
_D = 1024
_B = 4 * 8192
_NW = 32
_BPW = _B // _NW
_K = 32
_NCH = _BPW // _K

_mesh = plsc.VectorSubcoreMesh(core_axis_name="c", subcore_axis_name="s")


@functools.partial(
    pl.kernel,
    mesh=_mesh,
    out_type=jax.ShapeDtypeStruct((_B, _D), jnp.float32),
    scratch_types=[
        pltpu.VMEM((_NCH, _K), jnp.int32),
        pltpu.VMEM((2, _K, _D), jnp.float32),
        pltpu.SemaphoreType.DMA,
        pltpu.SemaphoreType.DMA,
    ],
)
def _emb_lookup(idx_hbm, table_hbm, out_hbm, idx_v, buf_v, sem_g, sem_o):
    wid = lax.axis_index("s") * 2 + lax.axis_index("c")
    base = wid * _BPW
    pltpu.sync_copy(idx_hbm.at[wid], idx_v)

    def body(ci, _):
        slot = lax.rem(ci, 2)
        pltpu.async_copy(table_hbm.at[idx_v.at[ci]], buf_v.at[slot], sem_g).wait()
        pltpu.sync_copy(buf_v.at[slot], out_hbm.at[pl.ds(base + ci * _K, _K)])
        return ()

    lax.fori_loop(0, _NCH, body, (), unroll=False)


def kernel(structural_position_ids, weight):
    ids = structural_position_ids.reshape(_NW, _NCH, _K).astype(jnp.int32)
    out = _emb_lookup(ids, weight)
    return out.reshape(structural_position_ids.shape + (_D,))
